# baseline (speedup 1.0000x reference)
import functools

import jax
import jax.numpy as jnp
from jax import lax
from jax.experimental import pallas as pl
from jax.experimental.pallas import tpu as pltpu
from jax.experimental.pallas import tpu_sc as plsc

NUM_EMBEDDINGS = 1000000
EMBEDDING_DIM = 16

_NI = 16384
_NJ = 200
_B = _NI * _NJ
_NW = 32
_C = 1024
_IBS = _NI // _C
_U = _B // (_NW * _C)
_SG = 2
_NS = _U // _SG
_NP = _U // 4
_RUN = 8 * _C


def _emb_kernel(x_hbm, w_hbm, out_hbm,
                ibuf0, ibuf1, rows0, rows1, rows2, rows3, t0, t1,
                sem_i0, sem_i1, sem_g0, sem_g1, sem_g2, sem_g3,
                sem_s0, sem_s1):
    wid = lax.axis_index("s") * 2 + lax.axis_index("c")

    ibuf = (ibuf0, ibuf1)
    rows = (rows0, rows1, rows2, rows3)
    t = (t0, t1)
    sem_i = (sem_i0, sem_i1)
    sem_g = (sem_g0, sem_g1, sem_g2, sem_g3)
    sem_s = (sem_s0, sem_s1)

    iota16 = lax.iota(jnp.int32, 16)

    def ibuf_copy(s, sb):
        q0 = (wid * _NS + s) * (_SG * _C)
        return pltpu.make_async_copy(
            x_hbm.at[pl.ds(q0, _SG * _C)], ibuf[sb], sem_i[sb])

    def gather(k):
        return pltpu.make_async_copy(
            w_hbm.at[ibuf[k // 2].at[pl.ds((k % 2) * _C, _C)]],
            rows[k], sem_g[k])

    def store(g, tb):
        u = wid * _U + g
        j = u // _IBS
        ib = u % _IBS
        return pltpu.make_async_copy(
            t[tb],
            out_hbm.at[pl.ds(2 * j, 2), pl.ds(ib * _RUN, _RUN)],
            sem_s[tb])

    def transpose(rk, tb):
        rb = rows[rk]
        tr = t[tb]

        def it_body(i_t, _):
            base = i_t * 128
            tbase = i_t * 1024
            for i_l0 in range(0, 128, 16):
                row_ids = base + i_l0 + iota16
                for d in range(EMBEDDING_DIM):
                    col_ids = jnp.full((16,), d, jnp.int32)
                    vals = plsc.load_gather(rb, [row_ids, col_ids])
                    tr[d // 8, pl.ds(tbase + (d % 8) * 128 + i_l0, 16)] = vals
            return 0

        lax.fori_loop(0, _C // 128, it_body, 0, unroll=False)

    ibuf_copy(0, 0).start()
    ibuf_copy(1, 1).start()
    ibuf_copy(0, 0).wait()
    gather(0).start()
    gather(1).start()

    def body(p, _):
        ibuf_copy(2 * p + 1, 1).wait()
        gather(2).start()
        gather(0).wait()

        @pl.when(p != 0)
        def _():
            store(4 * p - 2, 0).wait()
        transpose(0, 0)
        store(4 * p, 0).start()

        gather(3).start()
        gather(1).wait()

        @pl.when(p != _NP - 1)
        def _():
            ibuf_copy(2 * p + 2, 0).start()

        @pl.when(p != 0)
        def _():
            store(4 * p - 1, 1).wait()
        transpose(1, 1)
        store(4 * p + 1, 1).start()

        @pl.when(p != _NP - 1)
        def _():
            ibuf_copy(2 * p + 2, 0).wait()
            gather(0).start()
        gather(2).wait()

        store(4 * p, 0).wait()
        transpose(2, 0)
        store(4 * p + 2, 0).start()

        @pl.when(p != _NP - 1)
        def _():
            gather(1).start()
        gather(3).wait()

        @pl.when(p != _NP - 1)
        def _():
            ibuf_copy(2 * p + 3, 1).start()

        store(4 * p + 1, 1).wait()
        transpose(3, 1)
        store(4 * p + 3, 1).start()
        return 0

    lax.fori_loop(0, _NP, body, 0, unroll=False)

    store(_U - 2, 0).wait()
    store(_U - 1, 1).wait()


@jax.jit
def _embedding_lookup(x_flat, weights):
    mesh = plsc.VectorSubcoreMesh(core_axis_name="c", subcore_axis_name="s")
    fn = functools.partial(
        pl.kernel,
        mesh=mesh,
        out_type=jax.ShapeDtypeStruct((2 * _NJ, 128 * _C), jnp.float32),
        scratch_types=[
            pltpu.VMEM((_SG * _C,), jnp.int32),
            pltpu.VMEM((_SG * _C,), jnp.int32),
            pltpu.VMEM((_C, EMBEDDING_DIM), jnp.float32),
            pltpu.VMEM((_C, EMBEDDING_DIM), jnp.float32),
            pltpu.VMEM((_C, EMBEDDING_DIM), jnp.float32),
            pltpu.VMEM((_C, EMBEDDING_DIM), jnp.float32),
            pltpu.VMEM((2, _RUN), jnp.float32),
            pltpu.VMEM((2, _RUN), jnp.float32),
            pltpu.SemaphoreType.DMA,
            pltpu.SemaphoreType.DMA,
            pltpu.SemaphoreType.DMA,
            pltpu.SemaphoreType.DMA,
            pltpu.SemaphoreType.DMA,
            pltpu.SemaphoreType.DMA,
            pltpu.SemaphoreType.DMA,
            pltpu.SemaphoreType.DMA,
        ],
        compiler_params=pltpu.CompilerParams(
            use_tc_tiling_on_sc=False, needs_layout_passes=False),
    )(_emb_kernel)
    return fn(x_flat, weights)


def kernel(x, weights):
    x_flat = x.T.reshape(-1)
    w128 = weights.reshape(NUM_EMBEDDINGS // 8, 8 * EMBEDDING_DIM)
    w128 = jax.lax.optimization_barrier(w128)
    out2d = _embedding_lookup(x_flat, w128.reshape(NUM_EMBEDDINGS, EMBEDDING_DIM))
    a5 = out2d.reshape(_NJ, 2, 128, 8, 128)
    return a5.transpose(2, 4, 0, 1, 3).reshape(_NI, _NJ, EMBEDDING_DIM)


# speedup vs baseline: 1.0013x; 1.0013x over previous
import functools

import jax
import jax.numpy as jnp
from jax import lax
from jax.experimental import pallas as pl
from jax.experimental.pallas import tpu as pltpu
from jax.experimental.pallas import tpu_sc as plsc

NUM_EMBEDDINGS = 1000000
EMBEDDING_DIM = 16

_NI = 16384
_NJ = 200
_B = _NI * _NJ
_NW = 32
_C = 1024
_IBS = _NI // _C
_U = _B // (_NW * _C)
_SG = 2
_NS = _U // _SG
_RUN = 8 * _C


def _emb_kernel(x_hbm, w_hbm, out_hbm,
                ibuf0, ibuf1, rows0, rows1, t0, t1,
                sem_i0, sem_i1, sem_g0, sem_g1, sem_s0, sem_s1):
    wid = lax.axis_index("s") * 2 + lax.axis_index("c")

    ibuf = (ibuf0, ibuf1)
    rows = (rows0, rows1)
    t = (t0, t1)
    sem_i = (sem_i0, sem_i1)
    sem_g = (sem_g0, sem_g1)
    sem_s = (sem_s0, sem_s1)

    iota16 = lax.iota(jnp.int32, 16)

    def ibuf_copy(s, sb):
        q0 = (wid * _NS + s) * (_SG * _C)
        return pltpu.make_async_copy(
            x_hbm.at[pl.ds(q0, _SG * _C)], ibuf[sb], sem_i[sb])

    def gather(k, b):
        sb = (k // _SG) % 2
        return pltpu.make_async_copy(
            w_hbm.at[ibuf[sb].at[pl.ds((k % _SG) * _C, _C)]],
            rows[b], sem_g[b])

    def store(g, b):
        u = wid * _U + g
        j = u // _IBS
        ib = u % _IBS
        return pltpu.make_async_copy(
            t[b],
            out_hbm.at[pl.ds(2 * j, 2), pl.ds(ib * _RUN, _RUN)],
            sem_s[b])

    def transpose(b):
        rb = rows[b]
        tb = t[b]

        def it_body(i_t, _):
            base = i_t * 128
            tbase = i_t * 1024
            for i_l0 in range(0, 128, 16):
                row_ids = base + i_l0 + iota16
                for d in range(EMBEDDING_DIM):
                    col_ids = jnp.full((16,), d, jnp.int32)
                    vals = plsc.load_gather(rb, [row_ids, col_ids])
                    tb[d // 8, pl.ds(tbase + (d % 8) * 128 + i_l0, 16)] = vals
            return 0

        lax.fori_loop(0, _C // 128, it_body, 0, unroll=False)

    ibuf_copy(0, 0).start()
    ibuf_copy(1, 1).start()
    ibuf_copy(0, 0).wait()
    gather(0, 0).start()

    _NP = _U // 4

    def body(p, _):
        for k in range(4):
            g = p * 4 + k
            b = k % 2
            nb = 1 - b

            if k == 1:
                ibuf_copy(2 * p + 1, 1).wait()
                gather(2, nb).start()
            elif k == 3:
                @pl.when(p != _NP - 1)
                def _():
                    ibuf_copy(2 * p + 2, 0).wait()
                    gather(0, nb).start()
            else:
                gather(k + 1, nb).start()
            gather(k, b).wait()

            if k >= 2:
                store(g - 2, b).wait()
            else:
                @pl.when(p != 0)
                def _():
                    store(g - 2, b).wait()

            transpose(b)
            store(g, b).start()

            if k == 1:
                @pl.when(p < _NP - 1)
                def _():
                    ibuf_copy(2 * p + 2, 0).start()
            elif k == 3:
                @pl.when(p < _NP - 1)
                def _():
                    ibuf_copy(2 * p + 3, 1).start()
        return 0

    lax.fori_loop(0, _NP, body, 0, unroll=False)

    store(_U - 2, 0).wait()
    store(_U - 1, 1).wait()


@jax.jit
def _embedding_lookup(x_flat, weights):
    mesh = plsc.VectorSubcoreMesh(core_axis_name="c", subcore_axis_name="s")
    fn = functools.partial(
        pl.kernel,
        mesh=mesh,
        out_type=jax.ShapeDtypeStruct((2 * _NJ, 128 * _C), jnp.float32),
        scratch_types=[
            pltpu.VMEM((_SG * _C,), jnp.int32),
            pltpu.VMEM((_SG * _C,), jnp.int32),
            pltpu.VMEM((_C, EMBEDDING_DIM), jnp.float32),
            pltpu.VMEM((_C, EMBEDDING_DIM), jnp.float32),
            pltpu.VMEM((2, _RUN), jnp.float32),
            pltpu.VMEM((2, _RUN), jnp.float32),
            pltpu.SemaphoreType.DMA,
            pltpu.SemaphoreType.DMA,
            pltpu.SemaphoreType.DMA,
            pltpu.SemaphoreType.DMA,
            pltpu.SemaphoreType.DMA,
            pltpu.SemaphoreType.DMA,
        ],
        compiler_params=pltpu.CompilerParams(
            use_tc_tiling_on_sc=False, needs_layout_passes=False),
    )(_emb_kernel)
    return fn(x_flat, weights)


def kernel(x, weights):
    x_flat = x.T.reshape(-1)
    out2d = _embedding_lookup(x_flat, weights)
    a5 = out2d.reshape(_NJ, 2, 128, 8, 128)
    return a5.transpose(2, 4, 0, 1, 3).reshape(_NI, _NJ, EMBEDDING_DIM)
